# baseline (speedup 1.0000x reference)
import functools

import jax
import jax.numpy as jnp
from jax import lax
from jax.experimental import pallas as pl
from jax.experimental.pallas import tpu as pltpu
from jax.experimental.pallas import tpu_sc as plsc

_V = 100000
_D = 64
_B = 4096
_C = 20

_NC = 2
_NS = 16
_NW = _NC * _NS
_RPW = _B // _NW
_RCHUNK = 4
_NCHUNK = _RPW // _RCHUNK

_BN = 512
_NBLK = (_V + _BN - 1) // _BN


@functools.partial(
    pl.kernel,
    mesh=plsc.VectorSubcoreMesh(core_axis_name="c", subcore_axis_name="s"),
    compiler_params=pltpu.CompilerParams(use_tc_tiling_on_sc=False),
    out_type=jax.ShapeDtypeStruct((_B, _D), jnp.float32),
    scratch_types=[
        pltpu.VMEM((_RCHUNK * _C,), jnp.int32),
        pltpu.VMEM((_RCHUNK * _C, _D), jnp.float32),
        pltpu.VMEM((_RCHUNK, _D), jnp.float32),
        pltpu.SemaphoreType.DMA,
    ],
)
def _sc_gather_mean(ctx_hbm, tab_hbm, out_hbm, idx_v, rows_v, acc_v, sem):
    wid = lax.axis_index("s") * _NC + lax.axis_index("c")
    row0 = wid * _RPW

    def chunk(i, carry):
        r0 = row0 + i * _RCHUNK
        pltpu.sync_copy(ctx_hbm.at[pl.ds(r0 * _C, _RCHUNK * _C)], idx_v)
        pltpu.async_copy(tab_hbm.at[idx_v], rows_v, sem).wait()
        for r in range(_RCHUNK):
            for d in range(_D // 16):
                acc = rows_v[r * _C, pl.ds(d * 16, 16)]
                for c in range(1, _C):
                    acc = acc + rows_v[r * _C + c, pl.ds(d * 16, 16)]
                acc_v[r, pl.ds(d * 16, 16)] = acc * (1.0 / _C)
        pltpu.sync_copy(acc_v, out_hbm.at[pl.ds(r0, _RCHUNK)])
        return carry

    lax.fori_loop(0, _NCHUNK, chunk, 0)


def _mm_body(avg_ref, w_ref, b_ref, out_ref):
    out_ref[...] = lax.dot_general(
        avg_ref[...], w_ref[...],
        dimension_numbers=(((1,), (1,)), ((), ())),
        preferred_element_type=jnp.float32,
    ) + b_ref[...]


def _tc_matmul(avg, W, b2):
    return pl.pallas_call(
        _mm_body,
        grid=(_NBLK,),
        in_specs=[
            pl.BlockSpec((_B, _D), lambda i: (0, 0)),
            pl.BlockSpec((_BN, _D), lambda i: (i, 0)),
            pl.BlockSpec((1, _BN), lambda i: (0, i)),
        ],
        out_specs=pl.BlockSpec((_B, _BN), lambda i: (0, i)),
        out_shape=jax.ShapeDtypeStruct((_B, _V), jnp.float32),
    )(avg, W, b2)


def kernel(context, emb_table, W, b):
    ctx_flat = context.reshape(-1).astype(jnp.int32)
    avg = _sc_gather_mean(ctx_flat, emb_table)
    return _tc_matmul(avg, W, b.reshape(1, _V))


# speedup vs baseline: 1.0053x; 1.0053x over previous
import functools

import jax
import jax.numpy as jnp
from jax import lax
from jax.experimental import pallas as pl
from jax.experimental.pallas import tpu as pltpu
from jax.experimental.pallas import tpu_sc as plsc

_V = 100000
_D = 64
_B = 4096
_C = 20

_NC = 2
_NS = 16
_NW = _NC * _NS
_RPW = _B // _NW
_RCHUNK = 4
_NCHUNK = _RPW // _RCHUNK

_BM = 1024
_BN = 4096
_NBM = _B // _BM
_NBN = (_V + _BN - 1) // _BN


@functools.partial(
    pl.kernel,
    mesh=plsc.VectorSubcoreMesh(core_axis_name="c", subcore_axis_name="s"),
    compiler_params=pltpu.CompilerParams(use_tc_tiling_on_sc=False),
    out_type=jax.ShapeDtypeStruct((_B, _D), jnp.float32),
    scratch_types=[
        pltpu.VMEM((_RCHUNK * _C,), jnp.int32),
        pltpu.VMEM((_RCHUNK * _C, _D), jnp.float32),
        pltpu.VMEM((_RCHUNK, _D), jnp.float32),
        pltpu.SemaphoreType.DMA,
    ],
)
def _sc_gather_mean(ctx_hbm, tab_hbm, out_hbm, idx_v, rows_v, acc_v, sem):
    wid = lax.axis_index("s") * _NC + lax.axis_index("c")
    row0 = wid * _RPW

    def chunk(i, carry):
        r0 = row0 + i * _RCHUNK
        pltpu.sync_copy(ctx_hbm.at[pl.ds(r0 * _C, _RCHUNK * _C)], idx_v)
        pltpu.async_copy(tab_hbm.at[idx_v], rows_v, sem).wait()
        for r in range(_RCHUNK):
            for d in range(_D // 16):
                acc = rows_v[r * _C, pl.ds(d * 16, 16)]
                for c in range(1, _C):
                    acc = acc + rows_v[r * _C + c, pl.ds(d * 16, 16)]
                acc_v[r, pl.ds(d * 16, 16)] = acc * (1.0 / _C)
        pltpu.sync_copy(acc_v, out_hbm.at[pl.ds(r0, _RCHUNK)])
        return carry

    lax.fori_loop(0, _NCHUNK, chunk, 0)


def _mm_body(avg_ref, w_ref, b_ref, out_ref):
    out_ref[...] = lax.dot_general(
        avg_ref[...], w_ref[...],
        dimension_numbers=(((1,), (1,)), ((), ())),
        preferred_element_type=jnp.float32,
    ) + b_ref[...]


def _tc_matmul(avg, W, b2):
    return pl.pallas_call(
        _mm_body,
        grid=(_NBN, _NBM),
        in_specs=[
            pl.BlockSpec((_BM, _D), lambda i, j: (j, 0)),
            pl.BlockSpec((_BN, _D), lambda i, j: (i, 0)),
            pl.BlockSpec((1, _BN), lambda i, j: (0, i)),
        ],
        out_specs=pl.BlockSpec((_BM, _BN), lambda i, j: (j, i)),
        out_shape=jax.ShapeDtypeStruct((_B, _V), jnp.float32),
    )(avg, W, b2)


def kernel(context, emb_table, W, b):
    ctx_flat = context.reshape(-1).astype(jnp.int32)
    avg = _sc_gather_mean(ctx_flat, emb_table)
    return _tc_matmul(avg, W, b.reshape(1, _V))


# speedup vs baseline: 3.0217x; 3.0058x over previous
import functools

import jax
import jax.numpy as jnp
from jax import lax
from jax.experimental import pallas as pl
from jax.experimental.pallas import tpu as pltpu
from jax.experimental.pallas import tpu_sc as plsc

_V = 100000
_D = 64
_B = 4096
_C = 20

_NC = 2
_NS = 16
_NW = _NC * _NS
_RPW = _B // _NW
_RCHUNK = 4
_NCHUNK = _RPW // _RCHUNK

_BN = 1024
_NBN = (_V + _BN - 1) // _BN


@functools.partial(
    pl.kernel,
    mesh=plsc.VectorSubcoreMesh(core_axis_name="c", subcore_axis_name="s"),
    compiler_params=pltpu.CompilerParams(use_tc_tiling_on_sc=False),
    out_type=jax.ShapeDtypeStruct((_B, _D), jnp.float32),
    scratch_types=[
        pltpu.VMEM((_RCHUNK * _C,), jnp.int32),
        pltpu.VMEM((_RCHUNK * _C, _D), jnp.float32),
        pltpu.VMEM((_RCHUNK, _D), jnp.float32),
        pltpu.SemaphoreType.DMA,
    ],
)
def _sc_gather_mean(ctx_hbm, tab_hbm, out_hbm, idx_v, rows_v, acc_v, sem):
    wid = lax.axis_index("s") * _NC + lax.axis_index("c")
    row0 = wid * _RPW

    def chunk(i, carry):
        r0 = row0 + i * _RCHUNK
        pltpu.sync_copy(ctx_hbm.at[pl.ds(r0 * _C, _RCHUNK * _C)], idx_v)
        pltpu.async_copy(tab_hbm.at[idx_v], rows_v, sem).wait()
        for r in range(_RCHUNK):
            for d in range(_D // 16):
                acc = rows_v[r * _C, pl.ds(d * 16, 16)]
                for c in range(1, _C):
                    acc = acc + rows_v[r * _C + c, pl.ds(d * 16, 16)]
                acc_v[r, pl.ds(d * 16, 16)] = acc * (1.0 / _C)
        pltpu.sync_copy(acc_v, out_hbm.at[pl.ds(r0, _RCHUNK)])
        return carry

    lax.fori_loop(0, _NCHUNK, chunk, 0)


def _mm_body(avg_ref, w_ref, b_ref, out_ref):
    out_ref[...] = lax.dot_general(
        w_ref[...], avg_ref[...],
        dimension_numbers=(((1,), (1,)), ((), ())),
        preferred_element_type=jnp.float32,
    ) + b_ref[...]


def _tc_matmul_t(avg, W, b_col):
    return pl.pallas_call(
        _mm_body,
        grid=(_NBN,),
        in_specs=[
            pl.BlockSpec((_B, _D), lambda i: (0, 0)),
            pl.BlockSpec((_BN, _D), lambda i: (i, 0)),
            pl.BlockSpec((_BN, 1), lambda i: (i, 0)),
        ],
        out_specs=pl.BlockSpec((_BN, _B), lambda i: (i, 0)),
        out_shape=jax.ShapeDtypeStruct((_V, _B), jnp.float32),
    )(avg, W, b_col)


def kernel(context, emb_table, W, b):
    ctx_flat = context.reshape(-1).astype(jnp.int32)
    avg = _sc_gather_mean(ctx_flat, emb_table)
    logits_t = _tc_matmul_t(avg, W, b.reshape(_V, 1))
    return logits_t.T


# speedup vs baseline: 3.2616x; 1.0794x over previous
import functools

import jax
import jax.numpy as jnp
from jax import lax
from jax.experimental import pallas as pl
from jax.experimental.pallas import tpu as pltpu
from jax.experimental.pallas import tpu_sc as plsc

_V = 100000
_D = 64
_B = 4096
_C = 20

_NC = 2
_NS = 16
_NW = _NC * _NS
_RPW = _B // _NW
_RCHUNK = 4
_NCHUNK = _RPW // _RCHUNK

_BN = 1024
_NBN = (_V + _BN - 1) // _BN


@functools.partial(
    pl.kernel,
    mesh=plsc.VectorSubcoreMesh(core_axis_name="c", subcore_axis_name="s"),
    compiler_params=pltpu.CompilerParams(use_tc_tiling_on_sc=False),
    out_type=jax.ShapeDtypeStruct((_B, _D), jnp.float32),
    scratch_types=[
        pltpu.VMEM((_RCHUNK * _C,), jnp.int32),
        pltpu.VMEM((_RCHUNK * _C, _D), jnp.float32),
        pltpu.VMEM((_RCHUNK, _D), jnp.float32),
        pltpu.SemaphoreType.DMA,
    ],
)
def _sc_gather_mean(ctx_hbm, tab_hbm, out_hbm, idx_v, rows_v, acc_v, sem):
    wid = lax.axis_index("s") * _NC + lax.axis_index("c")
    row0 = wid * _RPW

    def chunk(i, carry):
        r0 = row0 + i * _RCHUNK
        pltpu.sync_copy(ctx_hbm.at[pl.ds(r0 * _C, _RCHUNK * _C)], idx_v)
        pltpu.async_copy(tab_hbm.at[idx_v], rows_v, sem).wait()
        for r in range(_RCHUNK):
            for d in range(_D // 16):
                acc = rows_v[r * _C, pl.ds(d * 16, 16)]
                for c in range(1, _C):
                    acc = acc + rows_v[r * _C + c, pl.ds(d * 16, 16)]
                acc_v[r, pl.ds(d * 16, 16)] = acc * (1.0 / _C)
        pltpu.sync_copy(acc_v, out_hbm.at[pl.ds(r0, _RCHUNK)])
        return carry

    lax.fori_loop(0, _NCHUNK, chunk, 0)


def _mm_body(avg_ref, wt_ref, b_ref, out_ref):
    bias = lax.broadcast_in_dim(b_ref[0, 0, :], (_BN, _B), (0,))
    out_ref[...] = lax.dot_general(
        wt_ref[...], avg_ref[...],
        dimension_numbers=(((0,), (1,)), ((), ())),
        preferred_element_type=jnp.float32,
    ) + bias


def _tc_matmul_t(avg, W_t, b_pad):
    return pl.pallas_call(
        _mm_body,
        grid=(_NBN,),
        in_specs=[
            pl.BlockSpec((_B, _D), lambda i: (0, 0)),
            pl.BlockSpec((_D, _BN), lambda i: (0, i)),
            pl.BlockSpec((1, 1, _BN), lambda i: (i, 0, 0)),
        ],
        out_specs=pl.BlockSpec((_BN, _B), lambda i: (i, 0)),
        out_shape=jax.ShapeDtypeStruct((_V, _B), jnp.float32),
    )(avg, W_t, b_pad)


def kernel(context, emb_table, W, b):
    ctx_flat = context.reshape(-1).astype(jnp.int32)
    avg = _sc_gather_mean(ctx_flat, emb_table)
    b_pad = jnp.pad(b, (0, _NBN * _BN - _V)).reshape(_NBN, 1, _BN)
    logits_t = _tc_matmul_t(avg, W.T, b_pad)
    return logits_t.T


# speedup vs baseline: 3.4088x; 1.0451x over previous
import functools

import jax
import jax.numpy as jnp
from jax import lax
from jax.experimental import pallas as pl
from jax.experimental.pallas import tpu as pltpu
from jax.experimental.pallas import tpu_sc as plsc

_V = 100000
_D = 64
_B = 4096
_C = 20

_NC = 2
_NS = 16
_NW = _NC * _NS
_RPW = _B // _NW
_RCHUNK = 4
_NCHUNK = _RPW // _RCHUNK

_BN = 1024
_NBN = (_V + _BN - 1) // _BN


@functools.partial(
    pl.kernel,
    mesh=plsc.VectorSubcoreMesh(core_axis_name="c", subcore_axis_name="s"),
    compiler_params=pltpu.CompilerParams(use_tc_tiling_on_sc=False),
    out_type=jax.ShapeDtypeStruct((_B, _D), jnp.float32),
    scratch_types=[
        pltpu.VMEM((_NCHUNK, _RCHUNK * _C), jnp.int32),
        pltpu.VMEM((_RCHUNK * _C, _D), jnp.float32),
        pltpu.VMEM((_RCHUNK * _C, _D), jnp.float32),
        pltpu.VMEM((_RPW, _D), jnp.float32),
        pltpu.SemaphoreType.DMA,
        pltpu.SemaphoreType.DMA,
    ],
)
def _sc_gather_mean(ctx2_hbm, tab_hbm, out_hbm, idx_v, bufa, bufb, out_v,
                    sema, semb):
    wid = lax.axis_index("s") * _NC + lax.axis_index("c")

    pltpu.sync_copy(ctx2_hbm.at[pl.ds(wid * _NCHUNK, _NCHUNK)], idx_v)

    def _fire(i, buf, sem):
        pltpu.make_async_copy(tab_hbm.at[idx_v.at[i]], buf, sem).start()

    def _drain(buf, sem):
        pltpu.make_async_copy(tab_hbm.at[idx_v.at[0]], buf, sem).wait()

    def _accum(chunk_i, buf):
        for r in range(_RCHUNK):
            for d in range(_D // 16):
                acc = buf[r * _C, pl.ds(d * 16, 16)]
                for c in range(1, _C):
                    acc = acc + buf[r * _C + c, pl.ds(d * 16, 16)]
                out_v[chunk_i * _RCHUNK + r, pl.ds(d * 16, 16)] = (
                    acc * (1.0 / _C))

    _fire(0, bufa, sema)

    def pair(g, carry):
        _fire(2 * g + 1, bufb, semb)
        _drain(bufa, sema)
        _accum(2 * g, bufa)

        @pl.when(g < _NCHUNK // 2 - 1)
        def _():
            _fire(2 * g + 2, bufa, sema)

        _drain(bufb, semb)
        _accum(2 * g + 1, bufb)
        return carry

    lax.fori_loop(0, _NCHUNK // 2, pair, 0)
    pltpu.sync_copy(out_v, out_hbm.at[pl.ds(wid * _RPW, _RPW)])


def _mm_body(avg_ref, wt_ref, b_ref, out_ref):
    bias = lax.broadcast_in_dim(b_ref[0, 0, :], (_BN, _B), (0,))
    out_ref[...] = lax.dot_general(
        wt_ref[...], avg_ref[...],
        dimension_numbers=(((0,), (1,)), ((), ())),
        preferred_element_type=jnp.float32,
    ) + bias


def _tc_matmul_t(avg, W_t, b_pad):
    return pl.pallas_call(
        _mm_body,
        grid=(_NBN,),
        in_specs=[
            pl.BlockSpec((_B, _D), lambda i: (0, 0)),
            pl.BlockSpec((_D, _BN), lambda i: (0, i)),
            pl.BlockSpec((1, 1, _BN), lambda i: (i, 0, 0)),
        ],
        out_specs=pl.BlockSpec((_BN, _B), lambda i: (i, 0)),
        out_shape=jax.ShapeDtypeStruct((_V, _B), jnp.float32),
    )(avg, W_t, b_pad)


def kernel(context, emb_table, W, b):
    ctx2 = context.reshape(_NW * _NCHUNK, _RCHUNK * _C).astype(jnp.int32)
    avg = _sc_gather_mean(ctx2, emb_table)
    b_pad = jnp.pad(b, (0, _NBN * _BN - _V)).reshape(_NBN, 1, _BN)
    logits_t = _tc_matmul_t(avg, W.T, b_pad)
    return logits_t.T
